# baseline (speedup 1.0000x reference)
import functools

import jax
import jax.numpy as jnp
from jax import lax
from jax.experimental import pallas as pl
from jax.experimental.pallas import tpu as pltpu
from jax.experimental.pallas import tpu_sc as plsc

D = 32


def _sc_gather(user_id, item_id, P, Q, U, V):
    info = plsc.get_sparse_core_info()
    nw = info.num_cores * info.num_subcores
    bsz = user_id.shape[0]
    b_per_w = bsz // nw
    nc = info.num_cores

    mesh = plsc.VectorSubcoreMesh(core_axis_name="c", subcore_axis_name="s")
    out_t = [jax.ShapeDtypeStruct((bsz, D), jnp.float32) for _ in range(4)]

    @functools.partial(
        pl.kernel,
        mesh=mesh,
        out_type=out_t,
        scratch_types=[
            pltpu.VMEM((b_per_w,), jnp.int32),
            pltpu.VMEM((b_per_w,), jnp.int32),
            pltpu.VMEM((b_per_w, D), jnp.float32),
            pltpu.VMEM((b_per_w, D), jnp.float32),
            pltpu.VMEM((b_per_w, D), jnp.float32),
            pltpu.VMEM((b_per_w, D), jnp.float32),
            pltpu.SemaphoreType.DMA,
        ],
        compiler_params=pltpu.CompilerParams(use_tc_tiling_on_sc=False),
    )
    def gather_kernel(uid_h, iid_h, p_h, q_h, u_h, v_h,
                      po, qo, uo, vo,
                      ui_v, ii_v, pv, qv, uv, vv, sem):
        wid = lax.axis_index("s") * nc + lax.axis_index("c")
        base = wid * b_per_w
        pltpu.sync_copy(uid_h.at[pl.ds(base, b_per_w)], ui_v)
        pltpu.sync_copy(iid_h.at[pl.ds(base, b_per_w)], ii_v)
        c1 = pltpu.async_copy(p_h.at[ui_v], pv, sem)
        c2 = pltpu.async_copy(q_h.at[ii_v], qv, sem)
        c3 = pltpu.async_copy(u_h.at[ui_v], uv, sem)
        c4 = pltpu.async_copy(v_h.at[ii_v], vv, sem)
        c1.wait()
        c2.wait()
        c3.wait()
        c4.wait()
        pltpu.sync_copy(pv, po.at[pl.ds(base, b_per_w)])
        pltpu.sync_copy(qv, qo.at[pl.ds(base, b_per_w)])
        pltpu.sync_copy(uv, uo.at[pl.ds(base, b_per_w)])
        pltpu.sync_copy(vv, vo.at[pl.ds(base, b_per_w)])

    return gather_kernel(user_id, item_id, P, Q, U, V)


def _tc_body(pmf_r, qmf_r, pmlp_r, qmlp_r,
             w1_r, b1_r, w2_r, b2_r, w3_r, b3_r, wp_r, out_r):
    h = jnp.concatenate([pmlp_r[...], qmlp_r[...]], axis=1)
    h = jnp.maximum(
        jnp.dot(h, w1_r[...], preferred_element_type=jnp.float32) + b1_r[...], 0.0)
    h = jnp.maximum(
        jnp.dot(h, w2_r[...], preferred_element_type=jnp.float32) + b2_r[...], 0.0)
    h = jnp.maximum(
        jnp.dot(h, w3_r[...], preferred_element_type=jnp.float32) + b3_r[...], 0.0)
    g = pmf_r[...] * qmf_r[...]
    z = jnp.concatenate([g, h], axis=1)
    out_r[...] = jnp.dot(z, wp_r[...], preferred_element_type=jnp.float32)


def _tc_dense(pmf, qmf, pmlp, qmlp, W1, b1, W2, b2, W3, b3, Wp):
    bsz = pmf.shape[0]
    blk = 2048
    grid = bsz // blk

    def row_spec():
        return pl.BlockSpec((blk, D), lambda i: (i, 0))

    def full_spec(shape):
        return pl.BlockSpec(shape, lambda i: tuple(0 for _ in shape))

    b1r = b1.reshape(1, -1)
    b2r = b2.reshape(1, -1)
    b3r = b3.reshape(1, -1)

    return pl.pallas_call(
        _tc_body,
        grid=(grid,),
        in_specs=[
            row_spec(), row_spec(), row_spec(), row_spec(),
            full_spec(W1.shape), full_spec(b1r.shape),
            full_spec(W2.shape), full_spec(b2r.shape),
            full_spec(W3.shape), full_spec(b3r.shape),
            full_spec(Wp.shape),
        ],
        out_specs=pl.BlockSpec((blk, 1), lambda i: (i, 0)),
        out_shape=jax.ShapeDtypeStruct((bsz, 1), jnp.float32),
    )(pmf, qmf, pmlp, qmlp, W1, b1r, W2, b2r, W3, b3r, Wp)


def kernel(user_id, item_id, P, Q, U, V, W1, b1, W2, b2, W3, b3, Wp):
    uid = user_id.astype(jnp.int32)
    iid = item_id.astype(jnp.int32)
    pmf, qmf, pmlp, qmlp = _sc_gather(uid, iid, P, Q, U, V)
    return _tc_dense(pmf, qmf, pmlp, qmlp, W1, b1, W2, b2, W3, b3, Wp)


# speedup vs baseline: 1.4183x; 1.4183x over previous
import functools

import jax
import jax.numpy as jnp
from jax import lax
from jax.experimental import pallas as pl
from jax.experimental.pallas import tpu as pltpu
from jax.experimental.pallas import tpu_sc as plsc

D = 32
CHUNK = 128


def _sc_gather(user_id, item_id, P, Q, U, V):
    info = plsc.get_sparse_core_info()
    nw = info.num_cores * info.num_subcores
    bsz = user_id.shape[0]
    b_per_w = bsz // nw
    nc = info.num_cores

    mesh = plsc.VectorSubcoreMesh(core_axis_name="c", subcore_axis_name="s")
    out_t = [jax.ShapeDtypeStruct((bsz, D), jnp.float32) for _ in range(4)]

    @functools.partial(
        pl.kernel,
        mesh=mesh,
        out_type=out_t,
        scratch_types=[
            pltpu.VMEM((b_per_w,), jnp.int32),
            pltpu.VMEM((b_per_w,), jnp.int32),
            pltpu.VMEM((CHUNK, D), jnp.float32),
            pltpu.VMEM((CHUNK, D), jnp.float32),
            pltpu.VMEM((CHUNK, D), jnp.float32),
            pltpu.VMEM((CHUNK, D), jnp.float32),
            pltpu.SemaphoreType.DMA,
        ],
    )
    def gather_kernel(uid_h, iid_h, p_h, q_h, u_h, v_h,
                      po, qo, uo, vo,
                      ui_v, ii_v, pv, qv, uv, vv, sem):
        wid = lax.axis_index("s") * nc + lax.axis_index("c")
        base = wid * b_per_w
        pltpu.sync_copy(uid_h.at[pl.ds(base, b_per_w)], ui_v)
        pltpu.sync_copy(iid_h.at[pl.ds(base, b_per_w)], ii_v)

        def chunk_body(c, _):
            c0 = c * CHUNK

            def fire(k, _):
                koff = k * 16
                uvec = ui_v[pl.ds(c0 + koff, 16)]
                tvec = ii_v[pl.ds(c0 + koff, 16)]
                for j in range(16):
                    u = uvec[j]
                    t = tvec[j]
                    pltpu.async_copy(p_h.at[u], pv.at[koff + j], sem)
                    pltpu.async_copy(u_h.at[u], uv.at[koff + j], sem)
                    pltpu.async_copy(q_h.at[t], qv.at[koff + j], sem)
                    pltpu.async_copy(v_h.at[t], vv.at[koff + j], sem)
                return 0

            lax.fori_loop(0, CHUNK // 16, fire, 0)

            def drain(i, _):
                pltpu.make_async_copy(p_h.at[0], pv.at[i], sem).wait()
                pltpu.make_async_copy(u_h.at[0], uv.at[i], sem).wait()
                pltpu.make_async_copy(q_h.at[0], qv.at[i], sem).wait()
                pltpu.make_async_copy(v_h.at[0], vv.at[i], sem).wait()
                return 0

            lax.fori_loop(0, CHUNK, drain, 0)

            pltpu.sync_copy(pv, po.at[pl.ds(base + c0, CHUNK)])
            pltpu.sync_copy(qv, qo.at[pl.ds(base + c0, CHUNK)])
            pltpu.sync_copy(uv, uo.at[pl.ds(base + c0, CHUNK)])
            pltpu.sync_copy(vv, vo.at[pl.ds(base + c0, CHUNK)])
            return 0

        lax.fori_loop(0, b_per_w // CHUNK, chunk_body, 0)

    return gather_kernel(user_id, item_id, P, Q, U, V)


def _tc_body(pmf_r, qmf_r, pmlp_r, qmlp_r,
             w1_r, b1_r, w2_r, b2_r, w3_r, b3_r, wp_r, out_r):
    h = jnp.concatenate([pmlp_r[...], qmlp_r[...]], axis=1)
    h = jnp.maximum(
        jnp.dot(h, w1_r[...], preferred_element_type=jnp.float32) + b1_r[...], 0.0)
    h = jnp.maximum(
        jnp.dot(h, w2_r[...], preferred_element_type=jnp.float32) + b2_r[...], 0.0)
    h = jnp.maximum(
        jnp.dot(h, w3_r[...], preferred_element_type=jnp.float32) + b3_r[...], 0.0)
    g = pmf_r[...] * qmf_r[...]
    z = jnp.concatenate([g, h], axis=1)
    out_r[...] = jnp.dot(z, wp_r[...], preferred_element_type=jnp.float32)


def _tc_dense(pmf, qmf, pmlp, qmlp, W1, b1, W2, b2, W3, b3, Wp):
    bsz = pmf.shape[0]
    blk = 2048
    grid = bsz // blk

    def row_spec():
        return pl.BlockSpec((blk, D), lambda i: (i, 0))

    def full_spec(shape):
        return pl.BlockSpec(shape, lambda i: tuple(0 for _ in shape))

    b1r = b1.reshape(1, -1)
    b2r = b2.reshape(1, -1)
    b3r = b3.reshape(1, -1)

    return pl.pallas_call(
        _tc_body,
        grid=(grid,),
        in_specs=[
            row_spec(), row_spec(), row_spec(), row_spec(),
            full_spec(W1.shape), full_spec(b1r.shape),
            full_spec(W2.shape), full_spec(b2r.shape),
            full_spec(W3.shape), full_spec(b3r.shape),
            full_spec(Wp.shape),
        ],
        out_specs=pl.BlockSpec((blk, 1), lambda i: (i, 0)),
        out_shape=jax.ShapeDtypeStruct((bsz, 1), jnp.float32),
    )(pmf, qmf, pmlp, qmlp, W1, b1r, W2, b2r, W3, b3r, Wp)


def kernel(user_id, item_id, P, Q, U, V, W1, b1, W2, b2, W3, b3, Wp):
    uid = user_id.astype(jnp.int32)
    iid = item_id.astype(jnp.int32)
    pmf, qmf, pmlp, qmlp = _sc_gather(uid, iid, P, Q, U, V)
    return _tc_dense(pmf, qmf, pmlp, qmlp, W1, b1, W2, b2, W3, b3, Wp)
